# baseline (speedup 1.0000x reference)
import functools

import jax
import jax.numpy as jnp
from jax import lax
from jax.experimental import pallas as pl
from jax.experimental.pallas import tpu as pltpu
from jax.experimental.pallas import tpu_sc as plsc

NUM_ENTITIES = 1000000
NUM_RELATIONS = 1000
D = 64
DP = 128
B = 16384

NC = 2
NS = 16
L = 16
NW = NC * NS
BPW = B // NW
OWN_SHIFT = 15
OWN = 1 << OWN_SHIFT
CW = 512
STRIP = 8 * CW
NB = B + L
WAVE2 = 128
SEG = 8192
ISL = 2048
TAIL0 = (NUM_ENTITIES // DP) * DP

_mesh = plsc.VectorSubcoreMesh(core_axis_name="c", subcore_axis_name="s")


@functools.partial(
    pl.kernel,
    mesh=_mesh,
    out_type=(jax.ShapeDtypeStruct((NB, DP), jnp.float32),
              jax.ShapeDtypeStruct((NB, DP), jnp.float32)),
    compiler_params=pltpu.CompilerParams(needs_layout_passes=False),
    scratch_types=[
        pltpu.VMEM((2048,), jnp.int32),
        pltpu.VMEM((B,), jnp.int32),
        pltpu.VMEM((B,), jnp.int32),
        pltpu.VMEM((D, CW), jnp.float32),
        pltpu.VMEM((D, CW), jnp.float32),
        pltpu.VMEM((SEG,), jnp.int32),
        pltpu.VMEM((L, DP), jnp.float32),
        pltpu.VMEM((L, DP), jnp.float32),
        pltpu.VMEM((DP - D, DP), jnp.float32),
        pltpu.SemaphoreType.DMA,
        pltpu.SemaphoreType.DMA,
        pltpu.SemaphoreType.DMA,
    ],
)
def _p1_scan(nodeT_hbm, tail_hbm, e_hbm, u_hbm, e_stage, u_stage,
             idxbuf, e_list, u_list, chunk_a, chunk_b, chits,
             colstage_a, colstage_b, tail_v, semc, sems, semi):
    wid = lax.axis_index("s") * NC + lax.axis_index("c")
    is_tail = wid == NW - 1
    lo = jnp.where(is_tail, TAIL0, wid << OWN_SHIFT)
    hi = jnp.where(is_tail, NUM_ENTITIES, jnp.minimum(lo + OWN, TAIL0))
    span = jnp.maximum(hi - lo, 0)
    nch = (span + CW - 1) // CW

    lane = lax.iota(jnp.int32, L)
    dummy_rows = B + lane

    pltpu.make_async_copy(colstage_a, e_stage.at[dummy_rows], sems).start()
    pltpu.make_async_copy(colstage_b, e_stage.at[dummy_rows], sems).start()

    def fill_list(src, lst):
        def outer(j, off):
            pltpu.async_copy(src.at[pl.ds(j * ISL, ISL)], idxbuf, semi).wait()

            def fv(i, off2):
                v = idxbuf[pl.ds(i * L, L)]
                m = (v >= lo) & (v < hi)
                packed = ((lane + (j * ISL + i * L)) << OWN_SHIFT) | (v - lo)
                plsc.store_compressed(lst.at[pl.ds(off2, L)], packed, mask=m)
                return off2 + plsc.all_reduce_population_count(m)[0]

            return lax.fori_loop(0, ISL // L, fv, off, unroll=2)

        return lax.fori_loop(0, B // ISL, outer, 0)

    ce = fill_list(e_hbm, e_list)
    cu = fill_list(u_hbm, u_list)

    def process(lst, cnt, stage, w_lo, w_hi, gather_ref, gg0):
        nseg = (cnt + SEG - 1) // SEG

        def seg_body(s0, carry0):
            seg_lo = s0 * SEG
            seg_n = jnp.minimum(cnt - seg_lo, SEG)

            def cv(i, off2):
                v = lst[pl.ds(seg_lo + i * L, L)]
                valid = (i * L + lane) < seg_n
                loc = v & (OWN - 1)
                m = valid & (loc >= w_lo) & (loc < w_hi)
                plsc.store_compressed(chits.at[pl.ds(off2, L)], v, mask=m)
                return off2 + plsc.all_reduce_population_count(m)[0]

            nhits = lax.fori_loop(0, (seg_n + L - 1) // L, cv, 0)

            def grp(g, gg):
                v = chits[pl.ds(g * L, L)]
                m = (g * L + lane) < nhits
                loc = v & (OWN - 1)
                slot = jnp.where(m, v >> OWN_SHIFT, B + lane)
                c_loc = jnp.where(m, loc - w_lo, 0)

                def extract(csbuf):
                    for d in range(D):
                        dsplat = jnp.full((L,), d, jnp.int32)
                        col = plsc.load_gather(gather_ref, [dsplat, c_loc])
                        plsc.store_scatter(csbuf, [lane, dsplat], col)
                    pltpu.make_async_copy(csbuf, stage.at[slot], sems).start()

                @pl.when(gg % 2 == 0)
                def _():
                    pltpu.make_async_copy(
                        colstage_a, stage.at[slot], sems).wait()
                    extract(colstage_a)

                @pl.when(gg % 2 == 1)
                def _o():
                    pltpu.make_async_copy(
                        colstage_b, stage.at[slot], sems).wait()
                    extract(colstage_b)
                return gg + 1

            ngrp = (nhits + L - 1) // L
            return lax.fori_loop(0, ngrp, grp, carry0)

        return lax.fori_loop(0, nseg, seg_body, gg0)

    def chunk_copies(k, buf):
        cps = []
        c0 = pl.multiple_of(lo + k * CW, CW)
        for i in range(8):
            cps.append(pltpu.make_async_copy(
                nodeT_hbm.at[pl.ds(8 * i, 8), pl.ds(c0, CW)],
                buf.at[pl.ds(8 * i, 8), :], semc))
        return cps

    def start_chunk(k, buf):
        for c in chunk_copies(k, buf):
            c.start()

    def wait_chunk(k, buf):
        for c in chunk_copies(k, buf):
            c.wait()

    def do_chunk(k, buf, gg):
        w_lo = k * CW
        w_hi = jnp.minimum(w_lo + CW, span)
        gg = process(e_list, ce, e_stage, w_lo, w_hi, buf, gg)
        return process(u_list, cu, u_stage, w_lo, w_hi, buf, gg)

    @pl.when(jnp.logical_not(is_tail))
    def _main():
        start_chunk(0, chunk_a)

        def pair(p, gg):
            k0 = 2 * p
            k1 = 2 * p + 1

            @pl.when(k1 < nch)
            def _():
                start_chunk(k1, chunk_b)
            wait_chunk(k0, chunk_a)
            gg = do_chunk(k0, chunk_a, gg)

            def odd(gg2):
                @pl.when(k1 + 1 < nch)
                def _s():
                    start_chunk(k1 + 1, chunk_a)
                wait_chunk(k1, chunk_b)
                return do_chunk(k1, chunk_b, gg2)

            gg = lax.cond(k1 < nch, odd, lambda gg2: gg2, gg)
            return gg

        lax.fori_loop(0, (nch + 1) // 2, pair, 0)

    @pl.when(is_tail)
    def _tail():
        pltpu.sync_copy(tail_hbm, tail_v)

        def process_tail(lst, cnt, stage):
            def lv(i, carry2):
                v = lst[pl.ds(i * L, L)]
                valid = (i * L + lane) < cnt
                loc = v & (OWN - 1)
                m = valid & (loc < span)
                nhit = jnp.sum(m.astype(jnp.int32), axis=0)

                @pl.when(nhit > 0)
                def _():
                    slot = jnp.where(m, v >> OWN_SHIFT, B + lane)
                    row = jnp.where(m, loc, 0)
                    for d in range(D):
                        dsplat = jnp.full((L,), d, jnp.int32)
                        col = plsc.load_gather(tail_v, [row, dsplat])
                        plsc.store_scatter(colstage_a, [lane, dsplat], col)
                    pltpu.sync_copy(colstage_a, stage.at[slot])
                return carry2

            lax.fori_loop(0, (cnt + L - 1) // L, lv, 0)

        process_tail(e_list, ce, e_stage)
        process_tail(u_list, cu, u_stage)

    pltpu.make_async_copy(colstage_a, e_stage.at[dummy_rows], sems).wait()
    pltpu.make_async_copy(colstage_b, e_stage.at[dummy_rows], sems).wait()


@functools.partial(
    pl.kernel,
    mesh=_mesh,
    out_type=jax.ShapeDtypeStruct((B,), jnp.float32),
    compiler_params=pltpu.CompilerParams(needs_layout_passes=False),
    scratch_types=[
        pltpu.VMEM((BPW,), jnp.int32),
        pltpu.VMEM((WAVE2, DP), jnp.float32),
        pltpu.VMEM((WAVE2, DP), jnp.float32),
        pltpu.VMEM((WAVE2, DP), jnp.float32),
        pltpu.VMEM((BPW,), jnp.float32),
        pltpu.VMEM((L * BPW,), jnp.float32),
        pltpu.SemaphoreType.DMA,
    ],
)
def _p2_score(e_stage, u_stage, edge_hbm, p_hbm, out_hbm,
              p_idx, e_rows, u_rows, p_rows, out_v, trans, sem):
    wid = lax.axis_index("s") * NC + lax.axis_index("c")
    base = wid * BPW

    pltpu.sync_copy(p_hbm.at[pl.ds(base, BPW)], p_idx)

    lane_off = lax.iota(jnp.int32, L) * BPW

    for w in range(BPW // WAVE2):
        w0 = w * WAVE2
        copies = [
            pltpu.async_copy(e_stage.at[pl.ds(base + w0, WAVE2)], e_rows, sem),
            pltpu.async_copy(u_stage.at[pl.ds(base + w0, WAVE2)], u_rows, sem),
            pltpu.async_copy(edge_hbm.at[p_idx.at[pl.ds(w0, WAVE2)]], p_rows, sem),
        ]
        for c in copies:
            c.wait()

        def elem(b, carry):
            acc = jnp.zeros((L,), jnp.float32)
            for c in range(D // L):
                sl = pl.ds(c * L, L)
                acc = acc + e_rows[b, sl] * p_rows[b, sl] * u_rows[b, sl]
            plsc.store_scatter(trans, [lane_off + (w0 + b)], acc)
            return carry

        lax.fori_loop(0, WAVE2, elem, 0, unroll=4)

    def red(g, carry):
        col0 = g * L
        s = trans[pl.ds(col0, L)]
        for j in range(1, L):
            s = s + trans[pl.ds(j * BPW + col0, L)]
        out_v[pl.ds(col0, L)] = s
        return carry

    lax.fori_loop(0, BPW // L, red, 0, unroll=2)

    pltpu.sync_copy(out_v, out_hbm.at[pl.ds(base, BPW)])


def kernel(node_embeddings, edge_embeddings, e_idc, p_idc, u_idc):
    nodeT = node_embeddings.T
    tail128 = jnp.pad(node_embeddings[TAIL0:, :], ((0, 0), (0, DP - D)))
    edge128 = jnp.pad(edge_embeddings, ((0, 0), (0, DP - D)))
    e32 = e_idc.astype(jnp.int32)
    p32 = p_idc.astype(jnp.int32)
    u32 = u_idc.astype(jnp.int32)
    e_stage, u_stage = _p1_scan(nodeT, tail128, e32, u32)
    return _p2_score(e_stage, u_stage, edge128, p32)


# speedup vs baseline: 1.2639x; 1.2639x over previous
import functools

import jax
import jax.numpy as jnp
from jax import lax
from jax.experimental import pallas as pl
from jax.experimental.pallas import tpu as pltpu
from jax.experimental.pallas import tpu_sc as plsc

NUM_ENTITIES = 1000000
NUM_RELATIONS = 1000
D = 64
DP = 128
B = 16384

NC = 2
NS = 16
L = 16
NW = NC * NS
BPW = B // NW
OWN_SHIFT = 15
OWN = 1 << OWN_SHIFT
CW = 1024
STRIP = 8 * CW
NB = B + L
WAVE2 = 128
SEG = 8192
ISL = 2048
TAIL0 = (NUM_ENTITIES // DP) * DP

_mesh = plsc.VectorSubcoreMesh(core_axis_name="c", subcore_axis_name="s")


@functools.partial(
    pl.kernel,
    mesh=_mesh,
    out_type=(jax.ShapeDtypeStruct((NB, DP), jnp.float32),
              jax.ShapeDtypeStruct((NB, DP), jnp.float32)),
    compiler_params=pltpu.CompilerParams(needs_layout_passes=False),
    scratch_types=[
        pltpu.VMEM((2048,), jnp.int32),
        pltpu.VMEM((B,), jnp.int32),
        pltpu.VMEM((B,), jnp.int32),
        pltpu.VMEM((D, CW), jnp.float32),
        pltpu.VMEM((SEG,), jnp.int32),
        pltpu.VMEM((L, DP), jnp.float32),
        pltpu.VMEM((L, DP), jnp.float32),
        pltpu.VMEM((DP - D, DP), jnp.float32),
        pltpu.SemaphoreType.DMA,
        pltpu.SemaphoreType.DMA,
        pltpu.SemaphoreType.DMA,
    ],
)
def _p1_scan(nodeT_hbm, tail_hbm, e_hbm, u_hbm, e_stage, u_stage,
             idxbuf, e_list, u_list, chunk_a, chits,
             colstage_a, colstage_b, tail_v, semc, sems, semi):
    wid = lax.axis_index("s") * NC + lax.axis_index("c")
    is_tail = wid == NW - 1
    lo = jnp.where(is_tail, TAIL0, wid << OWN_SHIFT)
    hi = jnp.where(is_tail, NUM_ENTITIES, jnp.minimum(lo + OWN, TAIL0))
    span = jnp.maximum(hi - lo, 0)
    nch = (span + CW - 1) // CW

    lane = lax.iota(jnp.int32, L)
    dummy_rows = B + lane

    pltpu.make_async_copy(colstage_a, e_stage.at[dummy_rows], sems).start()
    pltpu.make_async_copy(colstage_b, e_stage.at[dummy_rows], sems).start()

    def fill_list(src, lst):
        def outer(j, off):
            pltpu.async_copy(src.at[pl.ds(j * ISL, ISL)], idxbuf, semi).wait()

            def fv(i, off2):
                v = idxbuf[pl.ds(i * L, L)]
                m = (v >= lo) & (v < hi)
                packed = ((lane + (j * ISL + i * L)) << OWN_SHIFT) | (v - lo)
                plsc.store_compressed(lst.at[pl.ds(off2, L)], packed, mask=m)
                return off2 + plsc.all_reduce_population_count(m)[0]

            return lax.fori_loop(0, ISL // L, fv, off, unroll=2)

        return lax.fori_loop(0, B // ISL, outer, 0)

    ce = fill_list(e_hbm, e_list)
    cu = fill_list(u_hbm, u_list)

    def process(lst, cnt, stage, w_lo, w_hi, shift, gather_ref):
        nseg = (cnt + SEG - 1) // SEG

        def seg_body(s0, carry0):
            seg_lo = s0 * SEG
            seg_n = jnp.minimum(cnt - seg_lo, SEG)

            def cv(i, off2):
                v = lst[pl.ds(seg_lo + i * L, L)]
                valid = (i * L + lane) < seg_n
                loc = v & (OWN - 1)
                m = valid & (loc >= w_lo) & (loc < w_hi)
                plsc.store_compressed(chits.at[pl.ds(off2, L)], v, mask=m)
                return off2 + plsc.all_reduce_population_count(m)[0]

            nhits = lax.fori_loop(0, (seg_n + L - 1) // L, cv, 0)

            def grp(g, carry2):
                v = chits[pl.ds(g * L, L)]
                m = (g * L + lane) < nhits
                loc = v & (OWN - 1)
                slot = jnp.where(m, v >> OWN_SHIFT, B + lane)
                c_loc = jnp.where(m, loc - shift, 0)

                def extract(csbuf):
                    for d in range(D):
                        dsplat = jnp.full((L,), d, jnp.int32)
                        col = plsc.load_gather(gather_ref, [dsplat, c_loc])
                        plsc.store_scatter(csbuf, [lane, dsplat], col)
                    pltpu.make_async_copy(csbuf, stage.at[slot], sems).start()

                @pl.when(g % 2 == 0)
                def _():
                    pltpu.make_async_copy(
                        colstage_a, stage.at[slot], sems).wait()
                    extract(colstage_a)

                @pl.when(g % 2 == 1)
                def _o():
                    pltpu.make_async_copy(
                        colstage_b, stage.at[slot], sems).wait()
                    extract(colstage_b)
                return carry2

            ngrp = (nhits + L - 1) // L
            lax.fori_loop(0, ngrp, grp, 0)
            return carry0

        lax.fori_loop(0, nseg, seg_body, 0)

    @pl.when(jnp.logical_not(is_tail))
    def _main():
        def chunk_body(k, carry):
            c0 = lo + k * CW
            dma_c0 = pl.multiple_of(jnp.minimum(c0, TAIL0 - CW), DP)
            cps = []
            for i in range(8):
                cps.append(pltpu.make_async_copy(
                    nodeT_hbm.at[pl.ds(8 * i, 8), pl.ds(dma_c0, CW)],
                    chunk_a.at[pl.ds(8 * i, 8), :], semc))
            for c in cps:
                c.start()
            for c in cps:
                c.wait()
            w_lo = k * CW
            w_hi = jnp.minimum(w_lo + CW, span)
            shift = dma_c0 - lo
            process(e_list, ce, e_stage, w_lo, w_hi, shift, chunk_a)
            process(u_list, cu, u_stage, w_lo, w_hi, shift, chunk_a)
            return carry

        lax.fori_loop(0, nch, chunk_body, 0)

    @pl.when(is_tail)
    def _tail():
        pltpu.sync_copy(tail_hbm, tail_v)

        def process_tail(lst, cnt, stage):
            def lv(i, carry2):
                v = lst[pl.ds(i * L, L)]
                valid = (i * L + lane) < cnt
                loc = v & (OWN - 1)
                m = valid & (loc < span)
                nhit = jnp.sum(m.astype(jnp.int32), axis=0)

                @pl.when(nhit > 0)
                def _():
                    slot = jnp.where(m, v >> OWN_SHIFT, B + lane)
                    row = jnp.where(m, loc, 0)
                    pltpu.make_async_copy(
                        colstage_a, stage.at[slot], sems).wait()
                    for d in range(D):
                        dsplat = jnp.full((L,), d, jnp.int32)
                        col = plsc.load_gather(tail_v, [row, dsplat])
                        plsc.store_scatter(colstage_a, [lane, dsplat], col)
                    pltpu.make_async_copy(
                        colstage_a, stage.at[slot], sems).start()
                return carry2

            lax.fori_loop(0, (cnt + L - 1) // L, lv, 0)

        process_tail(e_list, ce, e_stage)
        process_tail(u_list, cu, u_stage)

    pltpu.make_async_copy(colstage_a, e_stage.at[dummy_rows], sems).wait()
    pltpu.make_async_copy(colstage_b, e_stage.at[dummy_rows], sems).wait()


@functools.partial(
    pl.kernel,
    mesh=_mesh,
    out_type=jax.ShapeDtypeStruct((B,), jnp.float32),
    compiler_params=pltpu.CompilerParams(needs_layout_passes=False),
    scratch_types=[
        pltpu.VMEM((BPW,), jnp.int32),
        pltpu.VMEM((2, WAVE2, DP), jnp.float32),
        pltpu.VMEM((2, WAVE2, DP), jnp.float32),
        pltpu.VMEM((2, WAVE2, DP), jnp.float32),
        pltpu.VMEM((BPW,), jnp.float32),
        pltpu.VMEM((L * BPW,), jnp.float32),
        pltpu.SemaphoreType.DMA,
        pltpu.SemaphoreType.DMA,
    ],
)
def _p2_score(e_stage, u_stage, edge_hbm, p_hbm, out_hbm,
              p_idx, e_rows, u_rows, p_rows, out_v, trans, sem_a, sem_b):
    wid = lax.axis_index("s") * NC + lax.axis_index("c")
    base = wid * BPW

    pltpu.sync_copy(p_hbm.at[pl.ds(base, BPW)], p_idx)

    lane_off = lax.iota(jnp.int32, L) * BPW
    nw = BPW // WAVE2
    sems2 = (sem_a, sem_b)

    def wave_copies(w):
        w0 = w * WAVE2
        pp = w % 2
        return [
            pltpu.async_copy(e_stage.at[pl.ds(base + w0, WAVE2)],
                             e_rows.at[pp], sems2[pp]),
            pltpu.async_copy(u_stage.at[pl.ds(base + w0, WAVE2)],
                             u_rows.at[pp], sems2[pp]),
            pltpu.async_copy(edge_hbm.at[p_idx.at[pl.ds(w0, WAVE2)]],
                             p_rows.at[pp], sems2[pp]),
        ]

    pending = {0: wave_copies(0)}
    for w in range(nw):
        w0 = w * WAVE2
        pp = w % 2
        if w + 1 < nw:
            pending[w + 1] = wave_copies(w + 1)
        for c in pending.pop(w):
            c.wait()
        ew, uw, pw = e_rows.at[pp], u_rows.at[pp], p_rows.at[pp]

        def elem(b, carry):
            acc = jnp.zeros((L,), jnp.float32)
            for c in range(D // L):
                sl = pl.ds(c * L, L)
                acc = acc + ew[b, sl] * pw[b, sl] * uw[b, sl]
            plsc.store_scatter(trans, [lane_off + (w0 + b)], acc)
            return carry

        lax.fori_loop(0, WAVE2, elem, 0, unroll=4)

    def red(g, carry):
        col0 = g * L
        s = trans[pl.ds(col0, L)]
        for j in range(1, L):
            s = s + trans[pl.ds(j * BPW + col0, L)]
        out_v[pl.ds(col0, L)] = s
        return carry

    lax.fori_loop(0, BPW // L, red, 0, unroll=2)

    pltpu.sync_copy(out_v, out_hbm.at[pl.ds(base, BPW)])


def kernel(node_embeddings, edge_embeddings, e_idc, p_idc, u_idc):
    nodeT = node_embeddings.T
    tail128 = jnp.pad(node_embeddings[TAIL0:, :], ((0, 0), (0, DP - D)))
    edge128 = jnp.pad(edge_embeddings, ((0, 0), (0, DP - D)))
    e32 = e_idc.astype(jnp.int32)
    p32 = p_idc.astype(jnp.int32)
    u32 = u_idc.astype(jnp.int32)
    e_stage, u_stage = _p1_scan(nodeT, tail128, e32, u32)
    return _p2_score(e_stage, u_stage, edge128, p32)
